# Initial kernel scaffold; baseline (speedup 1.0000x reference)
import jax
import jax.numpy as jnp
from jax.experimental import pallas as pl


def kernel(x):
    raise NotImplementedError("write your pallas kernel here")


# speedup vs baseline: 3.2533x; 3.2533x over previous
import functools

import jax
import jax.numpy as jnp
from jax import lax
from jax.experimental import pallas as pl
from jax.experimental.pallas import tpu as pltpu
from jax.experimental.pallas import tpu_sc as plsc

EPS = 1e-5
NROW = 4096
NCOL = 4096
LANES = 16
NJ = NCOL // LANES
BUF_ROWS = 12

_MESH = plsc.VectorSubcoreMesh(core_axis_name="c", subcore_axis_name="s")


def _rsqrt(m2):
    i = lax.bitcast_convert_type(m2, jnp.int32)
    i = jnp.int32(0x5F3759DF) - (i >> 1)
    y = lax.bitcast_convert_type(i, jnp.float32)
    for _ in range(3):
        y = y * (1.5 - 0.5 * m2 * y * y)
    return y


def _compute(buf, segsize, nsegs):
    def jbody(j, carry):
        col = pl.ds(j * LANES, LANES)
        for s in range(nsegs):
            r0 = s * segsize
            vs = [buf[r0 + i, col] for i in range(segsize)]
            m2 = vs[0] * vs[0]
            for v in vs[1:]:
                m2 = m2 + v * v
            m2 = m2 + EPS
            r = _rsqrt(m2)
            f = (m2 * r - 1.0) * jnp.minimum(r, 1.0)
            for i, v in enumerate(vs):
                buf[r0 + i, col] = v * f
        return carry

    lax.fori_loop(0, NJ, jbody, 0)


def _body(x_hbm, o_hbm, buf0, buf1, is0, is1, os0, os1):
    w = lax.axis_index("s") * 2 + lax.axis_index("c")
    batch = w // 4
    q = w % 4
    base = batch * 512
    chunks = [(base + 8 * q, 8, 1, 8)]
    for c in range(4):
        chunks.append((base + 32 + 48 * q + 12 * c, 12, 3, 4))
    for c in range(8):
        chunks.append((base + 224 + 72 * q + 9 * c, 9, 9, 1))
    n = len(chunks)

    bufs = (buf0, buf1)
    isems = (is0, is1)
    osems = (os0, os1)

    def in_cp(i):
        row, k, _, _ = chunks[i]
        return pltpu.make_async_copy(
            x_hbm.at[pl.ds(row, k)], bufs[i % 2].at[pl.ds(0, k)], isems[i % 2]
        )

    def out_cp(i):
        row, k, _, _ = chunks[i]
        return pltpu.make_async_copy(
            bufs[i % 2].at[pl.ds(0, k)], o_hbm.at[pl.ds(row, k)], osems[i % 2]
        )

    in_cp(0).start()
    for i in range(n):
        if i + 1 < n:
            if i >= 1:
                out_cp(i - 1).wait()
            in_cp(i + 1).start()
        in_cp(i).wait()
        _compute(bufs[i % 2], chunks[i][2], chunks[i][3])
        out_cp(i).start()
    out_cp(n - 2).wait()
    out_cp(n - 1).wait()


_sc_call = functools.partial(
    pl.kernel,
    out_type=jax.ShapeDtypeStruct((NROW, NCOL), jnp.float32),
    mesh=_MESH,
    scratch_types=[
        pltpu.VMEM((BUF_ROWS, NCOL), jnp.float32),
        pltpu.VMEM((BUF_ROWS, NCOL), jnp.float32),
        pltpu.SemaphoreType.DMA,
        pltpu.SemaphoreType.DMA,
        pltpu.SemaphoreType.DMA,
        pltpu.SemaphoreType.DMA,
    ],
    compiler_params=pltpu.CompilerParams(use_tc_tiling_on_sc=False),
)(_body)


def kernel(x):
    out = _sc_call(x.reshape(NROW, NCOL))
    return out.reshape(x.shape)


# speedup vs baseline: 3.6019x; 1.1072x over previous
import functools

import jax
import jax.numpy as jnp
from jax import lax
from jax.experimental import pallas as pl
from jax.experimental.pallas import tpu as pltpu
from jax.experimental.pallas import tpu_sc as plsc

EPS = 1e-5
NROW = 4096
NCOL = 4096
LANES = 16
NJ = NCOL // LANES
BUF_ROWS = 12

_MESH = plsc.VectorSubcoreMesh(core_axis_name="c", subcore_axis_name="s")


def _rsqrt(m2):
    i = lax.bitcast_convert_type(m2, jnp.int32)
    i = jnp.int32(0x5F3759DF) - (i >> 1)
    y = lax.bitcast_convert_type(i, jnp.float32)
    for _ in range(2):
        y = y * (1.5 - 0.5 * m2 * y * y)
    return y


def _compute(buf, segsize, nsegs):
    @plsc.parallel_loop(0, NJ, unroll=4)
    def jbody(j):
        col = pl.ds(j * LANES, LANES)
        for s in range(nsegs):
            r0 = s * segsize
            vs = [buf[r0 + i, col] for i in range(segsize)]
            m2 = vs[0] * vs[0]
            for v in vs[1:]:
                m2 = m2 + v * v
            m2 = m2 + EPS
            r = _rsqrt(m2)
            f = (m2 * r - 1.0) * jnp.minimum(r, 1.0)
            for i, v in enumerate(vs):
                buf[r0 + i, col] = v * f


def _body(x_hbm, o_hbm, buf0, buf1, is0, is1, os0, os1):
    w = lax.axis_index("s") * 2 + lax.axis_index("c")
    batch = w // 4
    q = w % 4
    base = batch * 512
    chunks = [(base + 8 * q, 8, 1, 8)]
    for c in range(4):
        chunks.append((base + 32 + 48 * q + 12 * c, 12, 3, 4))
    for c in range(8):
        chunks.append((base + 224 + 72 * q + 9 * c, 9, 9, 1))
    n = len(chunks)

    bufs = (buf0, buf1)
    isems = (is0, is1)
    osems = (os0, os1)

    def in_cp(i):
        row, k, _, _ = chunks[i]
        return pltpu.make_async_copy(
            x_hbm.at[pl.ds(row, k)], bufs[i % 2].at[pl.ds(0, k)], isems[i % 2]
        )

    def out_cp(i):
        row, k, _, _ = chunks[i]
        return pltpu.make_async_copy(
            bufs[i % 2].at[pl.ds(0, k)], o_hbm.at[pl.ds(row, k)], osems[i % 2]
        )

    in_cp(0).start()
    for i in range(n):
        if i + 1 < n:
            if i >= 1:
                out_cp(i - 1).wait()
            in_cp(i + 1).start()
        in_cp(i).wait()
        _compute(bufs[i % 2], chunks[i][2], chunks[i][3])
        out_cp(i).start()
    out_cp(n - 2).wait()
    out_cp(n - 1).wait()


_sc_call = functools.partial(
    pl.kernel,
    out_type=jax.ShapeDtypeStruct((NROW, NCOL), jnp.float32),
    mesh=_MESH,
    scratch_types=[
        pltpu.VMEM((BUF_ROWS, NCOL), jnp.float32),
        pltpu.VMEM((BUF_ROWS, NCOL), jnp.float32),
        pltpu.SemaphoreType.DMA,
        pltpu.SemaphoreType.DMA,
        pltpu.SemaphoreType.DMA,
        pltpu.SemaphoreType.DMA,
    ],
    compiler_params=pltpu.CompilerParams(use_tc_tiling_on_sc=False),
)(_body)


def kernel(x):
    out = _sc_call(x.reshape(NROW, NCOL))
    return out.reshape(x.shape)
